# baseline (speedup 1.0000x reference)
import functools

import jax
import jax.numpy as jnp
from jax import lax
from jax.experimental import pallas as pl
from jax.experimental.pallas import tpu as pltpu
from jax.experimental.pallas import tpu_sc as plsc

N_EDGES = 320000
D_FEAT = 128
N_SEGMENTS = 10000

NUM_CORES = 2
NUM_SUBCORES = 16
NUM_TILES = NUM_CORES * NUM_SUBCORES
CHUNK = 128
NCHUNKS = N_EDGES // CHUNK
TC_CHUNKS = 500
SC_CHUNKS = NCHUNKS - TC_CHUNKS
BASE_CHUNKS = SC_CHUNKS // NUM_TILES
EXTRA_TILES = SC_CHUNKS % NUM_TILES
SEG_PAD = 10240
SEG_PER_TILE = SEG_PAD // NUM_SUBCORES
TC_BLOCK = 1280
TC_NBLKS = TC_CHUNKS * CHUNK // TC_BLOCK
TC_BLK0 = SC_CHUNKS * CHUNK // TC_BLOCK
TC_WIN = 128
assert BASE_CHUNKS % 2 == 0
assert SC_CHUNKS * CHUNK % TC_BLOCK == 0
assert TC_CHUNKS * CHUNK % TC_BLOCK == 0


def _sc_partial_sums(x3, idx3):
    mesh = plsc.VectorSubcoreMesh(core_axis_name="c", subcore_axis_name="s")

    @functools.partial(
        pl.kernel,
        out_type=jax.ShapeDtypeStruct((NUM_CORES, SEG_PAD, D_FEAT), jnp.float32),
        mesh=mesh,
        scratch_types=[
            pltpu.VMEM((BASE_CHUNKS, 1, CHUNK), jnp.int32),
            pltpu.VMEM((1, CHUNK), jnp.int32),
            pltpu.VMEM((CHUNK, D_FEAT), jnp.float32),
            pltpu.VMEM((CHUNK, D_FEAT), jnp.float32),
            pltpu.VMEM_SHARED((SEG_PAD, D_FEAT), jnp.float32),
            pltpu.SemaphoreType.DMA,
            pltpu.SemaphoreType.DMA,
        ],
    )
    def k(x_hbm, idx_hbm, out_hbm, idx_v, idx_tail, buf0, buf1, acc,
          sem0, sem1):
        c = lax.axis_index("c")
        s = lax.axis_index("s")
        tile = s * NUM_CORES + c

        @pl.loop(0, CHUNK)
        def _zrow(i):
            @pl.loop(0, D_FEAT, step=16)
            def _zlane(j):
                buf0[i, pl.ds(j, 16)] = jnp.zeros((16,), jnp.float32)

        seg_base = pl.multiple_of(s * SEG_PER_TILE, 8)

        @pl.loop(0, SEG_PER_TILE, step=CHUNK)
        def _clear(r):
            pltpu.sync_copy(buf0, acc.at[pl.ds(seg_base + r, CHUNK)])

        plsc.subcore_barrier()

        start = BASE_CHUNKS * tile + jnp.minimum(tile, EXTRA_TILES)

        pltpu.sync_copy(idx_hbm.at[pl.ds(start, BASE_CHUNKS)], idx_v)

        pltpu.make_async_copy(x_hbm.at[start], buf0, sem0).start()
        pltpu.make_async_copy(x_hbm.at[start + 1], buf1, sem1).start()

        @pl.loop(0, BASE_CHUNKS, step=2)
        def _chunk(g):
            for b, buf, sem in ((0, buf0, sem0), (1, buf1, sem1)):
                pltpu.make_async_copy(x_hbm.at[start + g + b], buf, sem).wait()
                pltpu.sync_copy(buf, acc.at[idx_v.at[g + b].at[0]], add=True)

                @pl.when(g + b + 2 < BASE_CHUNKS)
                def _prefetch(buf=buf, sem=sem, off=b + 2):
                    pltpu.make_async_copy(
                        x_hbm.at[start + g + off], buf, sem
                    ).start()

        @pl.when(tile < EXTRA_TILES)
        def _tail():
            j = start + BASE_CHUNKS
            pltpu.sync_copy(idx_hbm.at[j], idx_tail)
            pltpu.sync_copy(x_hbm.at[j], buf0)
            pltpu.sync_copy(buf0, acc.at[idx_tail.at[0]], add=True)

        plsc.subcore_barrier()

        pltpu.sync_copy(
            acc.at[pl.ds(seg_base, SEG_PER_TILE)],
            out_hbm.at[c].at[pl.ds(seg_base, SEG_PER_TILE)],
        )

    return k(x3, idx3)


def _tc_partial_sums(x3, idx_col3, idx_row3):
    def body(ir_vec, ir_row, xr, out_ref):
        i = pl.program_id(0)

        @pl.when(i == 0)
        def _init():
            out_ref[...] = jnp.zeros_like(out_ref)

        first = ir_row[0, 0, 0]
        last = ir_row[0, 0, TC_BLOCK - 1]
        base = pl.multiple_of(
            jnp.minimum((first // 8) * 8, SEG_PAD - TC_WIN), 8
        )
        iota = lax.broadcasted_iota(jnp.int32, (TC_WIN, 1), 0) + base
        oh = (iota == ir_vec[0]).astype(jnp.bfloat16)
        xb = xr[0].astype(jnp.bfloat16)
        part = lax.dot_general(
            oh, xb, (((1,), (0,)), ((), ())),
            preferred_element_type=jnp.float32,
        )

        @pl.when(last - base < TC_WIN)
        def _fast():
            out_ref[pl.ds(base, TC_WIN), :] += part

        @pl.when(last - base >= TC_WIN)
        def _slow():
            def row(r, _):
                s = ir_row[0, 0, r]
                out_ref[pl.ds(s, 1), :] += xr[0, pl.ds(r, 1), :]
                return 0
            lax.fori_loop(0, TC_BLOCK, row, 0)

    return pl.pallas_call(
        body,
        grid=(TC_NBLKS,),
        in_specs=[
            pl.BlockSpec((1, 1, TC_BLOCK), lambda i: (TC_BLK0 + i, 0, 0)),
            pl.BlockSpec(
                (1, 1, TC_BLOCK),
                lambda i: (TC_BLK0 + i, 0, 0),
                memory_space=pltpu.SMEM,
            ),
            pl.BlockSpec((1, TC_BLOCK, D_FEAT), lambda i: (TC_BLK0 + i, 0, 0)),
        ],
        out_specs=pl.BlockSpec((SEG_PAD, D_FEAT), lambda i: (0, 0)),
        out_shape=jax.ShapeDtypeStruct((SEG_PAD, D_FEAT), jnp.float32),
    )(idx_col3, idx_row3, x3)


def _combine(sc_partials, tc_partial):
    def body(p_ref, t_ref, o_ref):
        o_ref[...] = p_ref[0] + p_ref[1] + t_ref[...]

    blk = 1000

    return pl.pallas_call(
        body,
        grid=(N_SEGMENTS // blk,),
        in_specs=[
            pl.BlockSpec((NUM_CORES, blk, D_FEAT), lambda i: (0, i, 0)),
            pl.BlockSpec((blk, D_FEAT), lambda i: (i, 0)),
        ],
        out_specs=pl.BlockSpec((blk, D_FEAT), lambda i: (i, 0)),
        out_shape=jax.ShapeDtypeStruct((N_SEGMENTS, D_FEAT), jnp.float32),
    )(sc_partials, tc_partial)


def kernel(x, idx):
    idx = idx.astype(jnp.int32)
    nb = N_EDGES // TC_BLOCK
    idx_rows = idx.reshape(nb, 1, TC_BLOCK)
    tc_part = _tc_partial_sums(
        x.reshape(nb, TC_BLOCK, D_FEAT),
        idx_rows,
        idx_rows,
    )
    x3 = x.reshape(NCHUNKS, CHUNK, D_FEAT)
    idx3 = idx.reshape(NCHUNKS, 1, CHUNK)
    sc_partials = _sc_partial_sums(x3, idx3)
    return _combine(sc_partials, tc_part)


# speedup vs baseline: 1.0123x; 1.0123x over previous
import functools

import jax
import jax.numpy as jnp
from jax import lax
from jax.experimental import pallas as pl
from jax.experimental.pallas import tpu as pltpu
from jax.experimental.pallas import tpu_sc as plsc

N_EDGES = 320000
D_FEAT = 128
N_SEGMENTS = 10000

NUM_CORES = 2
NUM_SUBCORES = 16
NUM_TILES = NUM_CORES * NUM_SUBCORES
CHUNK = 128
NCHUNKS = N_EDGES // CHUNK
TC_CHUNKS = 500
SC_CHUNKS = NCHUNKS - TC_CHUNKS
BASE_CHUNKS = SC_CHUNKS // NUM_TILES
EXTRA_TILES = SC_CHUNKS % NUM_TILES
SEG_PAD = 10240
SEG_PER_TILE = SEG_PAD // NUM_SUBCORES
TC_BLOCK = 1280
TC_NBLKS = TC_CHUNKS * CHUNK // TC_BLOCK
TC_BLK0 = SC_CHUNKS * CHUNK // TC_BLOCK
TC_WIN = 128
assert BASE_CHUNKS % 2 == 0
assert SC_CHUNKS * CHUNK % TC_BLOCK == 0
assert TC_CHUNKS * CHUNK % TC_BLOCK == 0


def _sc_partial_sums(x3, idx3):
    mesh = plsc.VectorSubcoreMesh(core_axis_name="c", subcore_axis_name="s")

    @functools.partial(
        pl.kernel,
        out_type=jax.ShapeDtypeStruct((NUM_CORES, SEG_PAD, D_FEAT), jnp.float32),
        mesh=mesh,
        scratch_types=[
            pltpu.VMEM((BASE_CHUNKS, 1, CHUNK), jnp.int32),
            pltpu.VMEM((1, CHUNK), jnp.int32),
            pltpu.VMEM((CHUNK, D_FEAT), jnp.float32),
            pltpu.VMEM((CHUNK, D_FEAT), jnp.float32),
            pltpu.VMEM_SHARED((SEG_PAD, D_FEAT), jnp.float32),
            pltpu.SemaphoreType.DMA,
            pltpu.SemaphoreType.DMA,
            pltpu.SemaphoreType.DMA,
        ],
    )
    def k(x_hbm, idx_hbm, out_hbm, idx_v, idx_tail, buf0, buf1, acc,
          sem0, sem1, semz):
        c = lax.axis_index("c")
        s = lax.axis_index("s")
        tile = s * NUM_CORES + c

        start = BASE_CHUNKS * tile + jnp.minimum(tile, EXTRA_TILES)

        pltpu.make_async_copy(
            idx_hbm.at[pl.ds(start, BASE_CHUNKS)], idx_v, semz
        ).start()
        pltpu.make_async_copy(x_hbm.at[start + 1], buf1, sem1).start()

        @pl.loop(0, CHUNK)
        def _zrow(i):
            @pl.loop(0, D_FEAT, step=16)
            def _zlane(j):
                buf0[i, pl.ds(j, 16)] = jnp.zeros((16,), jnp.float32)

        seg_base = pl.multiple_of(s * SEG_PER_TILE, 8)

        @pl.loop(0, SEG_PER_TILE, step=CHUNK)
        def _clear(r):
            pltpu.sync_copy(buf0, acc.at[pl.ds(seg_base + r, CHUNK)])

        pltpu.make_async_copy(x_hbm.at[start], buf0, sem0).start()
        pltpu.make_async_copy(
            idx_hbm.at[pl.ds(start, BASE_CHUNKS)], idx_v, semz
        ).wait()
        plsc.subcore_barrier()

        @pl.loop(0, BASE_CHUNKS, step=2)
        def _chunk(g):
            for b, buf, sem in ((0, buf0, sem0), (1, buf1, sem1)):
                pltpu.make_async_copy(x_hbm.at[start + g + b], buf, sem).wait()
                pltpu.sync_copy(buf, acc.at[idx_v.at[g + b].at[0]], add=True)

                @pl.when(g + b + 2 < BASE_CHUNKS)
                def _prefetch(buf=buf, sem=sem, off=b + 2):
                    pltpu.make_async_copy(
                        x_hbm.at[start + g + off], buf, sem
                    ).start()

        @pl.when(tile < EXTRA_TILES)
        def _tail():
            j = start + BASE_CHUNKS
            pltpu.sync_copy(idx_hbm.at[j], idx_tail)
            pltpu.sync_copy(x_hbm.at[j], buf0)
            pltpu.sync_copy(buf0, acc.at[idx_tail.at[0]], add=True)

        plsc.subcore_barrier()

        pltpu.sync_copy(
            acc.at[pl.ds(seg_base, SEG_PER_TILE)],
            out_hbm.at[c].at[pl.ds(seg_base, SEG_PER_TILE)],
        )

    return k(x3, idx3)


def _tc_partial_sums(x3, idx_col3, idx_row3):
    def body(ir_vec, ir_row, xr, out_ref):
        i = pl.program_id(0)

        @pl.when(i == 0)
        def _init():
            out_ref[...] = jnp.zeros_like(out_ref)

        first = ir_row[0, 0, 0]
        last = ir_row[0, 0, TC_BLOCK - 1]
        base = pl.multiple_of(
            jnp.minimum((first // 8) * 8, SEG_PAD - TC_WIN), 8
        )
        iota = lax.broadcasted_iota(jnp.int32, (TC_WIN, 1), 0) + base
        oh = (iota == ir_vec[0]).astype(jnp.bfloat16)
        xb = xr[0].astype(jnp.bfloat16)
        part = lax.dot_general(
            oh, xb, (((1,), (0,)), ((), ())),
            preferred_element_type=jnp.float32,
        )

        @pl.when(last - base < TC_WIN)
        def _fast():
            out_ref[pl.ds(base, TC_WIN), :] += part

        @pl.when(last - base >= TC_WIN)
        def _slow():
            def row(r, _):
                s = ir_row[0, 0, r]
                out_ref[pl.ds(s, 1), :] += xr[0, pl.ds(r, 1), :]
                return 0
            lax.fori_loop(0, TC_BLOCK, row, 0)

    return pl.pallas_call(
        body,
        grid=(TC_NBLKS,),
        in_specs=[
            pl.BlockSpec((1, 1, TC_BLOCK), lambda i: (TC_BLK0 + i, 0, 0)),
            pl.BlockSpec(
                (1, 1, TC_BLOCK),
                lambda i: (TC_BLK0 + i, 0, 0),
                memory_space=pltpu.SMEM,
            ),
            pl.BlockSpec((1, TC_BLOCK, D_FEAT), lambda i: (TC_BLK0 + i, 0, 0)),
        ],
        out_specs=pl.BlockSpec((SEG_PAD, D_FEAT), lambda i: (0, 0)),
        out_shape=jax.ShapeDtypeStruct((SEG_PAD, D_FEAT), jnp.float32),
    )(idx_col3, idx_row3, x3)


def _combine(sc_partials, tc_partial):
    def body(p_ref, t_ref, o_ref):
        o_ref[...] = p_ref[0] + p_ref[1] + t_ref[...]

    blk = 1000

    return pl.pallas_call(
        body,
        grid=(N_SEGMENTS // blk,),
        in_specs=[
            pl.BlockSpec((NUM_CORES, blk, D_FEAT), lambda i: (0, i, 0)),
            pl.BlockSpec((blk, D_FEAT), lambda i: (i, 0)),
        ],
        out_specs=pl.BlockSpec((blk, D_FEAT), lambda i: (i, 0)),
        out_shape=jax.ShapeDtypeStruct((N_SEGMENTS, D_FEAT), jnp.float32),
    )(sc_partials, tc_partial)


def kernel(x, idx):
    idx = idx.astype(jnp.int32)
    nb = N_EDGES // TC_BLOCK
    idx_rows = idx.reshape(nb, 1, TC_BLOCK)
    tc_part = _tc_partial_sums(
        x.reshape(nb, TC_BLOCK, D_FEAT),
        idx_rows,
        idx_rows,
    )
    x3 = x.reshape(NCHUNKS, CHUNK, D_FEAT)
    idx3 = idx.reshape(NCHUNKS, 1, CHUNK)
    sc_partials = _sc_partial_sums(x3, idx3)
    return _combine(sc_partials, tc_part)


# speedup vs baseline: 1.0501x; 1.0374x over previous
import functools

import jax
import jax.numpy as jnp
from jax import lax
from jax.experimental import pallas as pl
from jax.experimental.pallas import tpu as pltpu
from jax.experimental.pallas import tpu_sc as plsc

N_EDGES = 320000
D_FEAT = 128
N_SEGMENTS = 10000

NUM_CORES = 2
NUM_SUBCORES = 16
NUM_TILES = NUM_CORES * NUM_SUBCORES
CHUNK = 128
NCHUNKS = N_EDGES // CHUNK
TC_CHUNKS = 580
SC_CHUNKS = NCHUNKS - TC_CHUNKS
BASE_CHUNKS = SC_CHUNKS // NUM_TILES
EXTRA_TILES = SC_CHUNKS % NUM_TILES
SEG_PAD = 10240
SEG_PER_TILE = SEG_PAD // NUM_SUBCORES
TC_BLOCK = 1280
TC_NBLKS = TC_CHUNKS * CHUNK // TC_BLOCK
TC_BLK0 = SC_CHUNKS * CHUNK // TC_BLOCK
TC_WIN = 128
assert BASE_CHUNKS % 2 == 0
assert SC_CHUNKS * CHUNK % TC_BLOCK == 0
assert TC_CHUNKS * CHUNK % TC_BLOCK == 0


def _sc_partial_sums(x3, idx3):
    mesh = plsc.VectorSubcoreMesh(core_axis_name="c", subcore_axis_name="s")

    @functools.partial(
        pl.kernel,
        out_type=jax.ShapeDtypeStruct((NUM_CORES, SEG_PAD, D_FEAT), jnp.float32),
        mesh=mesh,
        scratch_types=[
            pltpu.VMEM((BASE_CHUNKS, 1, CHUNK), jnp.int32),
            pltpu.VMEM((1, CHUNK), jnp.int32),
            pltpu.VMEM((CHUNK, D_FEAT), jnp.float32),
            pltpu.VMEM((CHUNK, D_FEAT), jnp.float32),
            pltpu.VMEM_SHARED((SEG_PAD, D_FEAT), jnp.float32),
            pltpu.SemaphoreType.DMA,
            pltpu.SemaphoreType.DMA,
            pltpu.SemaphoreType.DMA,
        ],
    )
    def k(x_hbm, idx_hbm, out_hbm, idx_v, idx_tail, buf0, buf1, acc,
          sem0, sem1, semz):
        c = lax.axis_index("c")
        s = lax.axis_index("s")
        tile = s * NUM_CORES + c

        start = BASE_CHUNKS * tile + jnp.minimum(tile, EXTRA_TILES)

        pltpu.make_async_copy(
            idx_hbm.at[pl.ds(start, BASE_CHUNKS)], idx_v, semz
        ).start()
        pltpu.make_async_copy(x_hbm.at[start + 1], buf1, sem1).start()

        @pl.loop(0, CHUNK)
        def _zrow(i):
            @pl.loop(0, D_FEAT, step=16)
            def _zlane(j):
                buf0[i, pl.ds(j, 16)] = jnp.zeros((16,), jnp.float32)

        seg_base = pl.multiple_of(s * SEG_PER_TILE, 8)

        @pl.loop(0, SEG_PER_TILE, step=CHUNK)
        def _clear(r):
            pltpu.sync_copy(buf0, acc.at[pl.ds(seg_base + r, CHUNK)])

        pltpu.make_async_copy(x_hbm.at[start], buf0, sem0).start()
        pltpu.make_async_copy(
            idx_hbm.at[pl.ds(start, BASE_CHUNKS)], idx_v, semz
        ).wait()
        plsc.subcore_barrier()

        @pl.loop(0, BASE_CHUNKS, step=2)
        def _chunk(g):
            for b, buf, sem in ((0, buf0, sem0), (1, buf1, sem1)):
                pltpu.make_async_copy(x_hbm.at[start + g + b], buf, sem).wait()
                pltpu.sync_copy(buf, acc.at[idx_v.at[g + b].at[0]], add=True)

                @pl.when(g + b + 2 < BASE_CHUNKS)
                def _prefetch(buf=buf, sem=sem, off=b + 2):
                    pltpu.make_async_copy(
                        x_hbm.at[start + g + off], buf, sem
                    ).start()

        @pl.when(tile < EXTRA_TILES)
        def _tail():
            j = start + BASE_CHUNKS
            pltpu.sync_copy(idx_hbm.at[j], idx_tail)
            pltpu.sync_copy(x_hbm.at[j], buf0)
            pltpu.sync_copy(buf0, acc.at[idx_tail.at[0]], add=True)

        plsc.subcore_barrier()

        pltpu.sync_copy(
            acc.at[pl.ds(seg_base, SEG_PER_TILE)],
            out_hbm.at[c].at[pl.ds(seg_base, SEG_PER_TILE)],
        )

    return k(x3, idx3)


def _tc_partial_sums(x3, idx_col3, idx_row3):
    def body(ir_vec, ir_row, xr, out_ref):
        i = pl.program_id(0)

        @pl.when(i == 0)
        def _init():
            out_ref[...] = jnp.zeros_like(out_ref)

        first = ir_row[0, 0, 0]
        last = ir_row[0, 0, TC_BLOCK - 1]
        base = pl.multiple_of(
            jnp.minimum((first // 8) * 8, SEG_PAD - TC_WIN), 8
        )
        iota = lax.broadcasted_iota(jnp.int32, (TC_WIN, 1), 0) + base
        oh = (iota == ir_vec[0]).astype(jnp.bfloat16)
        xb = xr[0].astype(jnp.bfloat16)
        part = lax.dot_general(
            oh, xb, (((1,), (0,)), ((), ())),
            preferred_element_type=jnp.float32,
        )

        @pl.when(last - base < TC_WIN)
        def _fast():
            out_ref[pl.ds(base, TC_WIN), :] += part

        @pl.when(last - base >= TC_WIN)
        def _slow():
            def row(r, _):
                s = ir_row[0, 0, r]
                out_ref[pl.ds(s, 1), :] += xr[0, pl.ds(r, 1), :]
                return 0
            lax.fori_loop(0, TC_BLOCK, row, 0)

    return pl.pallas_call(
        body,
        grid=(TC_NBLKS,),
        in_specs=[
            pl.BlockSpec((1, 1, TC_BLOCK), lambda i: (TC_BLK0 + i, 0, 0)),
            pl.BlockSpec(
                (1, 1, TC_BLOCK),
                lambda i: (TC_BLK0 + i, 0, 0),
                memory_space=pltpu.SMEM,
            ),
            pl.BlockSpec((1, TC_BLOCK, D_FEAT), lambda i: (TC_BLK0 + i, 0, 0)),
        ],
        out_specs=pl.BlockSpec((SEG_PAD, D_FEAT), lambda i: (0, 0)),
        out_shape=jax.ShapeDtypeStruct((SEG_PAD, D_FEAT), jnp.float32),
    )(idx_col3, idx_row3, x3)


def _combine(sc_partials, tc_partial):
    def body(p_ref, t_ref, o_ref):
        o_ref[...] = p_ref[0] + p_ref[1] + t_ref[...]

    blk = 1000

    return pl.pallas_call(
        body,
        grid=(N_SEGMENTS // blk,),
        in_specs=[
            pl.BlockSpec((NUM_CORES, blk, D_FEAT), lambda i: (0, i, 0)),
            pl.BlockSpec((blk, D_FEAT), lambda i: (i, 0)),
        ],
        out_specs=pl.BlockSpec((blk, D_FEAT), lambda i: (i, 0)),
        out_shape=jax.ShapeDtypeStruct((N_SEGMENTS, D_FEAT), jnp.float32),
    )(sc_partials, tc_partial)


def kernel(x, idx):
    idx = idx.astype(jnp.int32)
    nb = N_EDGES // TC_BLOCK
    idx_rows = idx.reshape(nb, 1, TC_BLOCK)
    tc_part = _tc_partial_sums(
        x.reshape(nb, TC_BLOCK, D_FEAT),
        idx_rows,
        idx_rows,
    )
    x3 = x.reshape(NCHUNKS, CHUNK, D_FEAT)
    idx3 = idx.reshape(NCHUNKS, 1, CHUNK)
    sc_partials = _sc_partial_sums(x3, idx3)
    return _combine(sc_partials, tc_part)


# speedup vs baseline: 1.0519x; 1.0017x over previous
import functools

import jax
import jax.numpy as jnp
from jax import lax
from jax.experimental import pallas as pl
from jax.experimental.pallas import tpu as pltpu
from jax.experimental.pallas import tpu_sc as plsc

N_EDGES = 320000
D_FEAT = 128
N_SEGMENTS = 10000

NUM_CORES = 2
NUM_SUBCORES = 16
NUM_TILES = NUM_CORES * NUM_SUBCORES
CHUNK = 128
NCHUNKS = N_EDGES // CHUNK
TC_CHUNKS = 640
SC_CHUNKS = NCHUNKS - TC_CHUNKS
BASE_CHUNKS = SC_CHUNKS // NUM_TILES
EXTRA_TILES = SC_CHUNKS % NUM_TILES
SEG_PAD = 10240
SEG_PER_TILE = SEG_PAD // NUM_SUBCORES
TC_BLOCK = 1280
TC_NBLKS = TC_CHUNKS * CHUNK // TC_BLOCK
TC_BLK0 = SC_CHUNKS * CHUNK // TC_BLOCK
TC_WIN = 128
assert BASE_CHUNKS % 2 == 0
assert SC_CHUNKS * CHUNK % TC_BLOCK == 0
assert TC_CHUNKS * CHUNK % TC_BLOCK == 0


def _sc_partial_sums(x3, idx3):
    mesh = plsc.VectorSubcoreMesh(core_axis_name="c", subcore_axis_name="s")

    @functools.partial(
        pl.kernel,
        out_type=jax.ShapeDtypeStruct((NUM_CORES, SEG_PAD, D_FEAT), jnp.float32),
        mesh=mesh,
        scratch_types=[
            pltpu.VMEM((BASE_CHUNKS, 1, CHUNK), jnp.int32),
            pltpu.VMEM((1, CHUNK), jnp.int32),
            pltpu.VMEM((CHUNK, D_FEAT), jnp.float32),
            pltpu.VMEM((CHUNK, D_FEAT), jnp.float32),
            pltpu.VMEM_SHARED((SEG_PAD, D_FEAT), jnp.float32),
            pltpu.SemaphoreType.DMA,
            pltpu.SemaphoreType.DMA,
            pltpu.SemaphoreType.DMA,
        ],
    )
    def k(x_hbm, idx_hbm, out_hbm, idx_v, idx_tail, buf0, buf1, acc,
          sem0, sem1, semz):
        c = lax.axis_index("c")
        s = lax.axis_index("s")
        tile = s * NUM_CORES + c

        start = BASE_CHUNKS * tile + jnp.minimum(tile, EXTRA_TILES)

        pltpu.make_async_copy(
            idx_hbm.at[pl.ds(start, BASE_CHUNKS)], idx_v, semz
        ).start()
        pltpu.make_async_copy(x_hbm.at[start + 1], buf1, sem1).start()

        @pl.loop(0, CHUNK)
        def _zrow(i):
            @pl.loop(0, D_FEAT, step=16)
            def _zlane(j):
                buf0[i, pl.ds(j, 16)] = jnp.zeros((16,), jnp.float32)

        seg_base = pl.multiple_of(s * SEG_PER_TILE, 8)

        @pl.loop(0, SEG_PER_TILE, step=CHUNK)
        def _clear(r):
            pltpu.sync_copy(buf0, acc.at[pl.ds(seg_base + r, CHUNK)])

        pltpu.make_async_copy(x_hbm.at[start], buf0, sem0).start()
        pltpu.make_async_copy(
            idx_hbm.at[pl.ds(start, BASE_CHUNKS)], idx_v, semz
        ).wait()
        plsc.subcore_barrier()

        @pl.loop(0, BASE_CHUNKS, step=2)
        def _chunk(g):
            for b, buf, sem in ((0, buf0, sem0), (1, buf1, sem1)):
                pltpu.make_async_copy(x_hbm.at[start + g + b], buf, sem).wait()
                pltpu.sync_copy(buf, acc.at[idx_v.at[g + b].at[0]], add=True)

                @pl.when(g + b + 2 < BASE_CHUNKS)
                def _prefetch(buf=buf, sem=sem, off=b + 2):
                    pltpu.make_async_copy(
                        x_hbm.at[start + g + off], buf, sem
                    ).start()

        @pl.when(tile < EXTRA_TILES)
        def _tail():
            j = start + BASE_CHUNKS
            pltpu.sync_copy(idx_hbm.at[j], idx_tail)
            pltpu.sync_copy(x_hbm.at[j], buf0)
            pltpu.sync_copy(buf0, acc.at[idx_tail.at[0]], add=True)

        plsc.subcore_barrier()

        pltpu.sync_copy(
            acc.at[pl.ds(seg_base, SEG_PER_TILE)],
            out_hbm.at[c].at[pl.ds(seg_base, SEG_PER_TILE)],
        )

    return k(x3, idx3)


def _tc_partial_sums(x3, idx_col3, idx_row3):
    def body(ir_vec, ir_row, xr, out_ref):
        i = pl.program_id(0)

        @pl.when(i == 0)
        def _init():
            out_ref[...] = jnp.zeros_like(out_ref)

        first = ir_row[0, 0, 0]
        last = ir_row[0, 0, TC_BLOCK - 1]
        base = pl.multiple_of(
            jnp.minimum((first // 8) * 8, SEG_PAD - TC_WIN), 8
        )
        iota = lax.broadcasted_iota(jnp.int32, (TC_WIN, 1), 0) + base
        oh = (iota == ir_vec[0]).astype(jnp.bfloat16)
        xb = xr[0].astype(jnp.bfloat16)
        part = lax.dot_general(
            oh, xb, (((1,), (0,)), ((), ())),
            preferred_element_type=jnp.float32,
        )

        @pl.when(last - base < TC_WIN)
        def _fast():
            out_ref[pl.ds(base, TC_WIN), :] += part

        @pl.when(last - base >= TC_WIN)
        def _slow():
            def row(r, _):
                s = ir_row[0, 0, r]
                out_ref[pl.ds(s, 1), :] += xr[0, pl.ds(r, 1), :]
                return 0
            lax.fori_loop(0, TC_BLOCK, row, 0)

    return pl.pallas_call(
        body,
        grid=(TC_NBLKS,),
        in_specs=[
            pl.BlockSpec((1, 1, TC_BLOCK), lambda i: (TC_BLK0 + i, 0, 0)),
            pl.BlockSpec(
                (1, 1, TC_BLOCK),
                lambda i: (TC_BLK0 + i, 0, 0),
                memory_space=pltpu.SMEM,
            ),
            pl.BlockSpec((1, TC_BLOCK, D_FEAT), lambda i: (TC_BLK0 + i, 0, 0)),
        ],
        out_specs=pl.BlockSpec((SEG_PAD, D_FEAT), lambda i: (0, 0)),
        out_shape=jax.ShapeDtypeStruct((SEG_PAD, D_FEAT), jnp.float32),
    )(idx_col3, idx_row3, x3)


def _combine(sc_partials, tc_partial):
    def body(p_ref, t_ref, o_ref):
        o_ref[...] = p_ref[0] + p_ref[1] + t_ref[...]

    blk = 1000

    return pl.pallas_call(
        body,
        grid=(N_SEGMENTS // blk,),
        in_specs=[
            pl.BlockSpec((NUM_CORES, blk, D_FEAT), lambda i: (0, i, 0)),
            pl.BlockSpec((blk, D_FEAT), lambda i: (i, 0)),
        ],
        out_specs=pl.BlockSpec((blk, D_FEAT), lambda i: (i, 0)),
        out_shape=jax.ShapeDtypeStruct((N_SEGMENTS, D_FEAT), jnp.float32),
    )(sc_partials, tc_partial)


def kernel(x, idx):
    idx = idx.astype(jnp.int32)
    nb = N_EDGES // TC_BLOCK
    idx_rows = idx.reshape(nb, 1, TC_BLOCK)
    tc_part = _tc_partial_sums(
        x.reshape(nb, TC_BLOCK, D_FEAT),
        idx_rows,
        idx_rows,
    )
    x3 = x.reshape(NCHUNKS, CHUNK, D_FEAT)
    idx3 = idx.reshape(NCHUNKS, 1, CHUNK)
    sc_partials = _sc_partial_sums(x3, idx3)
    return _combine(sc_partials, tc_part)


# speedup vs baseline: 1.0713x; 1.0185x over previous
import functools

import jax
import jax.numpy as jnp
from jax import lax
from jax.experimental import pallas as pl
from jax.experimental.pallas import tpu as pltpu
from jax.experimental.pallas import tpu_sc as plsc

N_EDGES = 320000
D_FEAT = 128
N_SEGMENTS = 10000

NUM_CORES = 2
NUM_SUBCORES = 16
NUM_TILES = NUM_CORES * NUM_SUBCORES
CHUNK = 128
NCHUNKS = N_EDGES // CHUNK
TC_CHUNKS = 700
SC_CHUNKS = NCHUNKS - TC_CHUNKS
BASE_CHUNKS = SC_CHUNKS // NUM_TILES
EXTRA_TILES = SC_CHUNKS % NUM_TILES
SEG_PAD = 10240
SEG_PER_TILE = SEG_PAD // NUM_SUBCORES
TC_BLOCK = 1280
TC_NBLKS = TC_CHUNKS * CHUNK // TC_BLOCK
TC_BLK0 = SC_CHUNKS * CHUNK // TC_BLOCK
TC_WIN = 128
assert BASE_CHUNKS % 2 == 0
assert SC_CHUNKS * CHUNK % TC_BLOCK == 0
assert TC_CHUNKS * CHUNK % TC_BLOCK == 0


def _sc_partial_sums(x3, idx3):
    mesh = plsc.VectorSubcoreMesh(core_axis_name="c", subcore_axis_name="s")

    @functools.partial(
        pl.kernel,
        out_type=jax.ShapeDtypeStruct((NUM_CORES, SEG_PAD, D_FEAT), jnp.float32),
        mesh=mesh,
        scratch_types=[
            pltpu.VMEM((BASE_CHUNKS, 1, CHUNK), jnp.int32),
            pltpu.VMEM((1, CHUNK), jnp.int32),
            pltpu.VMEM((CHUNK, D_FEAT), jnp.float32),
            pltpu.VMEM((CHUNK, D_FEAT), jnp.float32),
            pltpu.VMEM_SHARED((SEG_PAD, D_FEAT), jnp.float32),
            pltpu.SemaphoreType.DMA,
            pltpu.SemaphoreType.DMA,
            pltpu.SemaphoreType.DMA,
        ],
    )
    def k(x_hbm, idx_hbm, out_hbm, idx_v, idx_tail, buf0, buf1, acc,
          sem0, sem1, semz):
        c = lax.axis_index("c")
        s = lax.axis_index("s")
        tile = s * NUM_CORES + c

        start = BASE_CHUNKS * tile + jnp.minimum(tile, EXTRA_TILES)

        pltpu.make_async_copy(
            idx_hbm.at[pl.ds(start, BASE_CHUNKS)], idx_v, semz
        ).start()
        pltpu.make_async_copy(x_hbm.at[start + 1], buf1, sem1).start()

        @pl.loop(0, CHUNK)
        def _zrow(i):
            @pl.loop(0, D_FEAT, step=16)
            def _zlane(j):
                buf0[i, pl.ds(j, 16)] = jnp.zeros((16,), jnp.float32)

        seg_base = pl.multiple_of(s * SEG_PER_TILE, 8)

        @pl.loop(0, SEG_PER_TILE, step=CHUNK)
        def _clear(r):
            pltpu.sync_copy(buf0, acc.at[pl.ds(seg_base + r, CHUNK)])

        pltpu.make_async_copy(x_hbm.at[start], buf0, sem0).start()
        pltpu.make_async_copy(
            idx_hbm.at[pl.ds(start, BASE_CHUNKS)], idx_v, semz
        ).wait()
        plsc.subcore_barrier()

        @pl.loop(0, BASE_CHUNKS, step=2)
        def _chunk(g):
            for b, buf, sem in ((0, buf0, sem0), (1, buf1, sem1)):
                pltpu.make_async_copy(x_hbm.at[start + g + b], buf, sem).wait()
                pltpu.sync_copy(buf, acc.at[idx_v.at[g + b].at[0]], add=True)

                @pl.when(g + b + 2 < BASE_CHUNKS)
                def _prefetch(buf=buf, sem=sem, off=b + 2):
                    pltpu.make_async_copy(
                        x_hbm.at[start + g + off], buf, sem
                    ).start()

        @pl.when(tile < EXTRA_TILES)
        def _tail():
            j = start + BASE_CHUNKS
            pltpu.sync_copy(idx_hbm.at[j], idx_tail)
            pltpu.sync_copy(x_hbm.at[j], buf0)
            pltpu.sync_copy(buf0, acc.at[idx_tail.at[0]], add=True)

        plsc.subcore_barrier()

        pltpu.sync_copy(
            acc.at[pl.ds(seg_base, SEG_PER_TILE)],
            out_hbm.at[c].at[pl.ds(seg_base, SEG_PER_TILE)],
        )

    return k(x3, idx3)


def _tc_partial_sums(x3, idx_col3, idx_row3):
    def body(ir_vec, ir_row, xr, out_ref):
        i = pl.program_id(0)

        @pl.when(i == 0)
        def _init():
            out_ref[...] = jnp.zeros_like(out_ref)

        first = ir_row[0, 0, 0]
        last = ir_row[0, 0, TC_BLOCK - 1]
        base = pl.multiple_of(
            jnp.minimum((first // 8) * 8, SEG_PAD - TC_WIN), 8
        )
        iota = lax.broadcasted_iota(jnp.int32, (TC_WIN, 1), 0) + base
        oh = (iota == ir_vec[0]).astype(jnp.bfloat16)
        xb = xr[0].astype(jnp.bfloat16)
        part = lax.dot_general(
            oh, xb, (((1,), (0,)), ((), ())),
            preferred_element_type=jnp.float32,
        )

        @pl.when(last - base < TC_WIN)
        def _fast():
            out_ref[pl.ds(base, TC_WIN), :] += part

        @pl.when(last - base >= TC_WIN)
        def _slow():
            def row(r, _):
                s = ir_row[0, 0, r]
                out_ref[pl.ds(s, 1), :] += xr[0, pl.ds(r, 1), :]
                return 0
            lax.fori_loop(0, TC_BLOCK, row, 0)

    return pl.pallas_call(
        body,
        grid=(TC_NBLKS,),
        in_specs=[
            pl.BlockSpec((1, 1, TC_BLOCK), lambda i: (TC_BLK0 + i, 0, 0)),
            pl.BlockSpec(
                (1, 1, TC_BLOCK),
                lambda i: (TC_BLK0 + i, 0, 0),
                memory_space=pltpu.SMEM,
            ),
            pl.BlockSpec((1, TC_BLOCK, D_FEAT), lambda i: (TC_BLK0 + i, 0, 0)),
        ],
        out_specs=pl.BlockSpec((SEG_PAD, D_FEAT), lambda i: (0, 0)),
        out_shape=jax.ShapeDtypeStruct((SEG_PAD, D_FEAT), jnp.float32),
    )(idx_col3, idx_row3, x3)


def _combine(sc_partials, tc_partial):
    def body(p_ref, t_ref, o_ref):
        o_ref[...] = p_ref[0] + p_ref[1] + t_ref[...]

    blk = 1000

    return pl.pallas_call(
        body,
        grid=(N_SEGMENTS // blk,),
        in_specs=[
            pl.BlockSpec((NUM_CORES, blk, D_FEAT), lambda i: (0, i, 0)),
            pl.BlockSpec((blk, D_FEAT), lambda i: (i, 0)),
        ],
        out_specs=pl.BlockSpec((blk, D_FEAT), lambda i: (i, 0)),
        out_shape=jax.ShapeDtypeStruct((N_SEGMENTS, D_FEAT), jnp.float32),
    )(sc_partials, tc_partial)


def kernel(x, idx):
    idx = idx.astype(jnp.int32)
    nb = N_EDGES // TC_BLOCK
    idx_rows = idx.reshape(nb, 1, TC_BLOCK)
    tc_part = _tc_partial_sums(
        x.reshape(nb, TC_BLOCK, D_FEAT),
        idx_rows,
        idx_rows,
    )
    x3 = x.reshape(NCHUNKS, CHUNK, D_FEAT)
    idx3 = idx.reshape(NCHUNKS, 1, CHUNK)
    sc_partials = _sc_partial_sums(x3, idx3)
    return _combine(sc_partials, tc_part)


# speedup vs baseline: 1.0969x; 1.0238x over previous
import functools

import jax
import jax.numpy as jnp
from jax import lax
from jax.experimental import pallas as pl
from jax.experimental.pallas import tpu as pltpu
from jax.experimental.pallas import tpu_sc as plsc

N_EDGES = 320000
D_FEAT = 128
N_SEGMENTS = 10000

NUM_CORES = 2
NUM_SUBCORES = 16
NUM_TILES = NUM_CORES * NUM_SUBCORES
CHUNK = 128
NCHUNKS = N_EDGES // CHUNK
TC_CHUNKS = 760
SC_CHUNKS = NCHUNKS - TC_CHUNKS
BASE_CHUNKS = SC_CHUNKS // NUM_TILES
EXTRA_TILES = SC_CHUNKS % NUM_TILES
SEG_PAD = 10240
SEG_PER_TILE = SEG_PAD // NUM_SUBCORES
TC_BLOCK = 1280
TC_NBLKS = TC_CHUNKS * CHUNK // TC_BLOCK
TC_BLK0 = SC_CHUNKS * CHUNK // TC_BLOCK
TC_WIN = 128
assert BASE_CHUNKS % 2 == 0
assert SC_CHUNKS * CHUNK % TC_BLOCK == 0
assert TC_CHUNKS * CHUNK % TC_BLOCK == 0


def _sc_partial_sums(x3, idx3):
    mesh = plsc.VectorSubcoreMesh(core_axis_name="c", subcore_axis_name="s")

    @functools.partial(
        pl.kernel,
        out_type=jax.ShapeDtypeStruct((NUM_CORES, SEG_PAD, D_FEAT), jnp.float32),
        mesh=mesh,
        scratch_types=[
            pltpu.VMEM((BASE_CHUNKS, 1, CHUNK), jnp.int32),
            pltpu.VMEM((1, CHUNK), jnp.int32),
            pltpu.VMEM((CHUNK, D_FEAT), jnp.float32),
            pltpu.VMEM((CHUNK, D_FEAT), jnp.float32),
            pltpu.VMEM_SHARED((SEG_PAD, D_FEAT), jnp.float32),
            pltpu.SemaphoreType.DMA,
            pltpu.SemaphoreType.DMA,
            pltpu.SemaphoreType.DMA,
        ],
    )
    def k(x_hbm, idx_hbm, out_hbm, idx_v, idx_tail, buf0, buf1, acc,
          sem0, sem1, semz):
        c = lax.axis_index("c")
        s = lax.axis_index("s")
        tile = s * NUM_CORES + c

        start = BASE_CHUNKS * tile + jnp.minimum(tile, EXTRA_TILES)

        pltpu.make_async_copy(
            idx_hbm.at[pl.ds(start, BASE_CHUNKS)], idx_v, semz
        ).start()
        pltpu.make_async_copy(x_hbm.at[start + 1], buf1, sem1).start()

        @pl.loop(0, CHUNK)
        def _zrow(i):
            @pl.loop(0, D_FEAT, step=16)
            def _zlane(j):
                buf0[i, pl.ds(j, 16)] = jnp.zeros((16,), jnp.float32)

        seg_base = pl.multiple_of(s * SEG_PER_TILE, 8)

        @pl.loop(0, SEG_PER_TILE, step=CHUNK)
        def _clear(r):
            pltpu.sync_copy(buf0, acc.at[pl.ds(seg_base + r, CHUNK)])

        pltpu.make_async_copy(x_hbm.at[start], buf0, sem0).start()
        pltpu.make_async_copy(
            idx_hbm.at[pl.ds(start, BASE_CHUNKS)], idx_v, semz
        ).wait()
        plsc.subcore_barrier()

        @pl.loop(0, BASE_CHUNKS, step=2)
        def _chunk(g):
            for b, buf, sem in ((0, buf0, sem0), (1, buf1, sem1)):
                pltpu.make_async_copy(x_hbm.at[start + g + b], buf, sem).wait()
                pltpu.sync_copy(buf, acc.at[idx_v.at[g + b].at[0]], add=True)

                @pl.when(g + b + 2 < BASE_CHUNKS)
                def _prefetch(buf=buf, sem=sem, off=b + 2):
                    pltpu.make_async_copy(
                        x_hbm.at[start + g + off], buf, sem
                    ).start()

        @pl.when(tile < EXTRA_TILES)
        def _tail():
            j = start + BASE_CHUNKS
            pltpu.sync_copy(idx_hbm.at[j], idx_tail)
            pltpu.sync_copy(x_hbm.at[j], buf0)
            pltpu.sync_copy(buf0, acc.at[idx_tail.at[0]], add=True)

        plsc.subcore_barrier()

        pltpu.sync_copy(
            acc.at[pl.ds(seg_base, SEG_PER_TILE)],
            out_hbm.at[c].at[pl.ds(seg_base, SEG_PER_TILE)],
        )

    return k(x3, idx3)


def _tc_partial_sums(x3, idx_col3, idx_row3):
    def body(ir_vec, ir_row, xr, out_ref):
        i = pl.program_id(0)

        @pl.when(i == 0)
        def _init():
            out_ref[...] = jnp.zeros_like(out_ref)

        first = ir_row[0, 0, 0]
        last = ir_row[0, 0, TC_BLOCK - 1]
        base = pl.multiple_of(
            jnp.minimum((first // 8) * 8, SEG_PAD - TC_WIN), 8
        )
        iota = lax.broadcasted_iota(jnp.int32, (TC_WIN, 1), 0) + base
        oh = (iota == ir_vec[0]).astype(jnp.bfloat16)
        xb = xr[0].astype(jnp.bfloat16)
        part = lax.dot_general(
            oh, xb, (((1,), (0,)), ((), ())),
            preferred_element_type=jnp.float32,
        )

        @pl.when(last - base < TC_WIN)
        def _fast():
            out_ref[pl.ds(base, TC_WIN), :] += part

        @pl.when(last - base >= TC_WIN)
        def _slow():
            def row(r, _):
                s = ir_row[0, 0, r]
                out_ref[pl.ds(s, 1), :] += xr[0, pl.ds(r, 1), :]
                return 0
            lax.fori_loop(0, TC_BLOCK, row, 0)

    return pl.pallas_call(
        body,
        grid=(TC_NBLKS,),
        in_specs=[
            pl.BlockSpec((1, 1, TC_BLOCK), lambda i: (TC_BLK0 + i, 0, 0)),
            pl.BlockSpec(
                (1, 1, TC_BLOCK),
                lambda i: (TC_BLK0 + i, 0, 0),
                memory_space=pltpu.SMEM,
            ),
            pl.BlockSpec((1, TC_BLOCK, D_FEAT), lambda i: (TC_BLK0 + i, 0, 0)),
        ],
        out_specs=pl.BlockSpec((SEG_PAD, D_FEAT), lambda i: (0, 0)),
        out_shape=jax.ShapeDtypeStruct((SEG_PAD, D_FEAT), jnp.float32),
    )(idx_col3, idx_row3, x3)


def _combine(sc_partials, tc_partial):
    def body(p_ref, t_ref, o_ref):
        o_ref[...] = p_ref[0] + p_ref[1] + t_ref[...]

    blk = 1000

    return pl.pallas_call(
        body,
        grid=(N_SEGMENTS // blk,),
        in_specs=[
            pl.BlockSpec((NUM_CORES, blk, D_FEAT), lambda i: (0, i, 0)),
            pl.BlockSpec((blk, D_FEAT), lambda i: (i, 0)),
        ],
        out_specs=pl.BlockSpec((blk, D_FEAT), lambda i: (i, 0)),
        out_shape=jax.ShapeDtypeStruct((N_SEGMENTS, D_FEAT), jnp.float32),
    )(sc_partials, tc_partial)


def kernel(x, idx):
    idx = idx.astype(jnp.int32)
    nb = N_EDGES // TC_BLOCK
    idx_rows = idx.reshape(nb, 1, TC_BLOCK)
    tc_part = _tc_partial_sums(
        x.reshape(nb, TC_BLOCK, D_FEAT),
        idx_rows,
        idx_rows,
    )
    x3 = x.reshape(NCHUNKS, CHUNK, D_FEAT)
    idx3 = idx.reshape(NCHUNKS, 1, CHUNK)
    sc_partials = _sc_partial_sums(x3, idx3)
    return _combine(sc_partials, tc_part)


# speedup vs baseline: 1.1180x; 1.0193x over previous
import functools

import jax
import jax.numpy as jnp
from jax import lax
from jax.experimental import pallas as pl
from jax.experimental.pallas import tpu as pltpu
from jax.experimental.pallas import tpu_sc as plsc

N_EDGES = 320000
D_FEAT = 128
N_SEGMENTS = 10000

NUM_CORES = 2
NUM_SUBCORES = 16
NUM_TILES = NUM_CORES * NUM_SUBCORES
CHUNK = 128
NCHUNKS = N_EDGES // CHUNK
TC_CHUNKS = 820
SC_CHUNKS = NCHUNKS - TC_CHUNKS
BASE_CHUNKS = SC_CHUNKS // NUM_TILES
EXTRA_TILES = SC_CHUNKS % NUM_TILES
SEG_PAD = 10240
SEG_PER_TILE = SEG_PAD // NUM_SUBCORES
TC_BLOCK = 1280
TC_NBLKS = TC_CHUNKS * CHUNK // TC_BLOCK
TC_BLK0 = SC_CHUNKS * CHUNK // TC_BLOCK
TC_WIN = 128
assert BASE_CHUNKS % 2 == 0
assert SC_CHUNKS * CHUNK % TC_BLOCK == 0
assert TC_CHUNKS * CHUNK % TC_BLOCK == 0


def _sc_partial_sums(x3, idx3):
    mesh = plsc.VectorSubcoreMesh(core_axis_name="c", subcore_axis_name="s")

    @functools.partial(
        pl.kernel,
        out_type=jax.ShapeDtypeStruct((NUM_CORES, SEG_PAD, D_FEAT), jnp.float32),
        mesh=mesh,
        scratch_types=[
            pltpu.VMEM((BASE_CHUNKS, 1, CHUNK), jnp.int32),
            pltpu.VMEM((1, CHUNK), jnp.int32),
            pltpu.VMEM((CHUNK, D_FEAT), jnp.float32),
            pltpu.VMEM((CHUNK, D_FEAT), jnp.float32),
            pltpu.VMEM_SHARED((SEG_PAD, D_FEAT), jnp.float32),
            pltpu.SemaphoreType.DMA,
            pltpu.SemaphoreType.DMA,
            pltpu.SemaphoreType.DMA,
        ],
    )
    def k(x_hbm, idx_hbm, out_hbm, idx_v, idx_tail, buf0, buf1, acc,
          sem0, sem1, semz):
        c = lax.axis_index("c")
        s = lax.axis_index("s")
        tile = s * NUM_CORES + c

        start = BASE_CHUNKS * tile + jnp.minimum(tile, EXTRA_TILES)

        pltpu.make_async_copy(
            idx_hbm.at[pl.ds(start, BASE_CHUNKS)], idx_v, semz
        ).start()
        pltpu.make_async_copy(x_hbm.at[start + 1], buf1, sem1).start()

        @pl.loop(0, CHUNK)
        def _zrow(i):
            @pl.loop(0, D_FEAT, step=16)
            def _zlane(j):
                buf0[i, pl.ds(j, 16)] = jnp.zeros((16,), jnp.float32)

        seg_base = pl.multiple_of(s * SEG_PER_TILE, 8)

        @pl.loop(0, SEG_PER_TILE, step=CHUNK)
        def _clear(r):
            pltpu.sync_copy(buf0, acc.at[pl.ds(seg_base + r, CHUNK)])

        pltpu.make_async_copy(x_hbm.at[start], buf0, sem0).start()
        pltpu.make_async_copy(
            idx_hbm.at[pl.ds(start, BASE_CHUNKS)], idx_v, semz
        ).wait()
        plsc.subcore_barrier()

        @pl.loop(0, BASE_CHUNKS, step=2)
        def _chunk(g):
            for b, buf, sem in ((0, buf0, sem0), (1, buf1, sem1)):
                pltpu.make_async_copy(x_hbm.at[start + g + b], buf, sem).wait()
                pltpu.sync_copy(buf, acc.at[idx_v.at[g + b].at[0]], add=True)

                @pl.when(g + b + 2 < BASE_CHUNKS)
                def _prefetch(buf=buf, sem=sem, off=b + 2):
                    pltpu.make_async_copy(
                        x_hbm.at[start + g + off], buf, sem
                    ).start()

        @pl.when(tile < EXTRA_TILES)
        def _tail():
            j = start + BASE_CHUNKS
            pltpu.sync_copy(idx_hbm.at[j], idx_tail)
            pltpu.sync_copy(x_hbm.at[j], buf0)
            pltpu.sync_copy(buf0, acc.at[idx_tail.at[0]], add=True)

        plsc.subcore_barrier()

        pltpu.sync_copy(
            acc.at[pl.ds(seg_base, SEG_PER_TILE)],
            out_hbm.at[c].at[pl.ds(seg_base, SEG_PER_TILE)],
        )

    return k(x3, idx3)


def _tc_partial_sums(x3, idx_col3, idx_row3):
    def body(ir_vec, ir_row, xr, out_ref):
        i = pl.program_id(0)

        @pl.when(i == 0)
        def _init():
            out_ref[...] = jnp.zeros_like(out_ref)

        first = ir_row[0, 0, 0]
        last = ir_row[0, 0, TC_BLOCK - 1]
        base = pl.multiple_of(
            jnp.minimum((first // 8) * 8, SEG_PAD - TC_WIN), 8
        )
        iota = lax.broadcasted_iota(jnp.int32, (TC_WIN, 1), 0) + base
        oh = (iota == ir_vec[0]).astype(jnp.bfloat16)
        xb = xr[0].astype(jnp.bfloat16)
        part = lax.dot_general(
            oh, xb, (((1,), (0,)), ((), ())),
            preferred_element_type=jnp.float32,
        )

        @pl.when(last - base < TC_WIN)
        def _fast():
            out_ref[pl.ds(base, TC_WIN), :] += part

        @pl.when(last - base >= TC_WIN)
        def _slow():
            def row(r, _):
                s = ir_row[0, 0, r]
                out_ref[pl.ds(s, 1), :] += xr[0, pl.ds(r, 1), :]
                return 0
            lax.fori_loop(0, TC_BLOCK, row, 0)

    return pl.pallas_call(
        body,
        grid=(TC_NBLKS,),
        in_specs=[
            pl.BlockSpec((1, 1, TC_BLOCK), lambda i: (TC_BLK0 + i, 0, 0)),
            pl.BlockSpec(
                (1, 1, TC_BLOCK),
                lambda i: (TC_BLK0 + i, 0, 0),
                memory_space=pltpu.SMEM,
            ),
            pl.BlockSpec((1, TC_BLOCK, D_FEAT), lambda i: (TC_BLK0 + i, 0, 0)),
        ],
        out_specs=pl.BlockSpec((SEG_PAD, D_FEAT), lambda i: (0, 0)),
        out_shape=jax.ShapeDtypeStruct((SEG_PAD, D_FEAT), jnp.float32),
    )(idx_col3, idx_row3, x3)


def _combine(sc_partials, tc_partial):
    def body(p_ref, t_ref, o_ref):
        o_ref[...] = p_ref[0] + p_ref[1] + t_ref[...]

    blk = 1000

    return pl.pallas_call(
        body,
        grid=(N_SEGMENTS // blk,),
        in_specs=[
            pl.BlockSpec((NUM_CORES, blk, D_FEAT), lambda i: (0, i, 0)),
            pl.BlockSpec((blk, D_FEAT), lambda i: (i, 0)),
        ],
        out_specs=pl.BlockSpec((blk, D_FEAT), lambda i: (i, 0)),
        out_shape=jax.ShapeDtypeStruct((N_SEGMENTS, D_FEAT), jnp.float32),
    )(sc_partials, tc_partial)


def kernel(x, idx):
    idx = idx.astype(jnp.int32)
    nb = N_EDGES // TC_BLOCK
    idx_rows = idx.reshape(nb, 1, TC_BLOCK)
    tc_part = _tc_partial_sums(
        x.reshape(nb, TC_BLOCK, D_FEAT),
        idx_rows,
        idx_rows,
    )
    x3 = x.reshape(NCHUNKS, CHUNK, D_FEAT)
    idx3 = idx.reshape(NCHUNKS, 1, CHUNK)
    sc_partials = _sc_partial_sums(x3, idx3)
    return _combine(sc_partials, tc_part)
